# baseline (speedup 1.0000x reference)
import jax
import jax.numpy as jnp
from jax.experimental import pallas as pl
from jax.experimental.pallas import tpu as pltpu

_ROWS = 128
_COLS = 32768
_BC = 8192
_NC = _BC // 128


def _scan_kernel(x_ref, o_ref, carry_ref):
    c = pl.program_id(0)

    @pl.when(c == 0)
    def _():
        carry_ref[...] = jnp.zeros_like(carry_ref)

    k = jax.lax.broadcasted_iota(jnp.int32, (128, 128), 0)
    j = jax.lax.broadcasted_iota(jnp.int32, (128, 128), 1)
    tri = (k <= j).astype(jnp.float32)

    running = carry_ref[...]
    for i in range(_NC):
        xi = x_ref[:, 128 * i:128 * (i + 1)]
        yi = jax.lax.dot_general(
            xi, tri, (((1,), (0,)), ((), ())),
            preferred_element_type=jnp.float32,
        )
        o_ref[:, 128 * i:128 * (i + 1)] = yi - xi + running
        running = running + yi[:, 127:128]
    carry_ref[...] = running


def kernel(x):
    grid = (_COLS // _BC,)
    return pl.pallas_call(
        _scan_kernel,
        grid=grid,
        in_specs=[pl.BlockSpec((_ROWS, _BC), lambda c: (0, c))],
        out_specs=pl.BlockSpec((_ROWS, _BC), lambda c: (0, c)),
        out_shape=jax.ShapeDtypeStruct((_ROWS, _COLS), jnp.float32),
        scratch_shapes=[pltpu.VMEM((_ROWS, 1), jnp.float32)],
    )(x)


# speedup vs baseline: 1.6223x; 1.6223x over previous
import jax
import jax.numpy as jnp
from jax.experimental import pallas as pl
from jax.experimental.pallas import tpu as pltpu

_ROWS = 128
_COLS = 32768
_BC = 8192
_NC = _BC // 128


def _scan_kernel(x_ref, o_ref, carry_ref, g_ref, m_ref, tri_ref):
    c = pl.program_id(0)

    @pl.when(c == 0)
    def _():
        carry_ref[...] = jnp.zeros_like(carry_ref)
        m = jax.lax.broadcasted_iota(jnp.int32, (_BC, 128), 0)
        n = jax.lax.broadcasted_iota(jnp.int32, (_BC, 128), 1)
        g_ref[...] = (((n < _NC) & (m // 128 == n - 1)) | (n == _NC)).astype(
            jnp.float32)
        k2 = jax.lax.broadcasted_iota(jnp.int32, (_NC, _BC), 0)
        j2 = jax.lax.broadcasted_iota(jnp.int32, (_NC, _BC), 1)
        m_ref[...] = (k2 <= j2 // 128).astype(jnp.float32)
        ks = jax.lax.broadcasted_iota(jnp.int32, (128, 128), 0)
        js = jax.lax.broadcasted_iota(jnp.int32, (128, 128), 1)
        tri_ref[...] = (ks < js).astype(jnp.float32)

    x = x_ref[...]
    carry = carry_ref[...]

    tots = jax.lax.dot_general(
        x, g_ref[...], (((1,), (0,)), ((), ())),
        preferred_element_type=jnp.float32)
    lane = jax.lax.broadcasted_iota(jnp.int32, (_ROWS, _NC), 1)
    taug = tots[:, :_NC] + jnp.where(lane == 0, carry, 0.0)

    offs = jax.lax.dot_general(
        taug, m_ref[...], (((1,), (0,)), ((), ())),
        preferred_element_type=jnp.float32)

    tri = tri_ref[...]
    for i in range(_NC):
        sl = slice(128 * i, 128 * (i + 1))
        yi = jax.lax.dot_general(
            x[:, sl], tri, (((1,), (0,)), ((), ())),
            preferred_element_type=jnp.float32)
        o_ref[:, sl] = yi + offs[:, sl]

    carry_ref[...] = carry + tots[:, _NC:_NC + 1]


def kernel(x):
    grid = (_COLS // _BC,)
    return pl.pallas_call(
        _scan_kernel,
        grid=grid,
        in_specs=[pl.BlockSpec((_ROWS, _BC), lambda c: (0, c))],
        out_specs=pl.BlockSpec((_ROWS, _BC), lambda c: (0, c)),
        out_shape=jax.ShapeDtypeStruct((_ROWS, _COLS), jnp.float32),
        scratch_shapes=[
            pltpu.VMEM((_ROWS, 1), jnp.float32),
            pltpu.VMEM((_BC, 128), jnp.float32),
            pltpu.VMEM((_NC, _BC), jnp.float32),
            pltpu.VMEM((128, 128), jnp.float32),
        ],
    )(x)


# speedup vs baseline: 1.6304x; 1.0050x over previous
import jax
import jax.numpy as jnp
from jax.experimental import pallas as pl
from jax.experimental.pallas import tpu as pltpu

_ROWS = 128
_COLS = 32768
_BC = 8192
_NC = _BC // 128


def _scan_kernel(x_ref, o_ref, carry_ref, m_ref, tri_ref):
    c = pl.program_id(0)

    @pl.when(c == 0)
    def _():
        carry_ref[...] = jnp.zeros_like(carry_ref)
        k2 = jax.lax.broadcasted_iota(jnp.int32, (_NC, _BC), 0)
        j2 = jax.lax.broadcasted_iota(jnp.int32, (_NC, _BC), 1)
        m_ref[...] = (k2 <= j2 // 128).astype(jnp.float32)
        ks = jax.lax.broadcasted_iota(jnp.int32, (128, 128), 0)
        js = jax.lax.broadcasted_iota(jnp.int32, (128, 128), 1)
        tri_ref[...] = (ks < js).astype(jnp.float32)

    x = x_ref[...]
    carry = carry_ref[...]

    tots = [jnp.sum(x[:, 128 * i:128 * (i + 1)], axis=1, keepdims=True)
            for i in range(_NC)]
    tcat = jnp.concatenate(tots, axis=1)
    taug = jnp.concatenate([carry, tcat[:, :-1]], axis=1)

    offs = jax.lax.dot_general(
        taug, m_ref[...], (((1,), (0,)), ((), ())),
        preferred_element_type=jnp.float32)

    tri = tri_ref[...]
    for i in range(_NC):
        sl = slice(128 * i, 128 * (i + 1))
        yi = jax.lax.dot_general(
            x[:, sl], tri, (((1,), (0,)), ((), ())),
            preferred_element_type=jnp.float32)
        o_ref[:, sl] = yi + offs[:, sl]

    carry_ref[...] = carry + jnp.sum(tcat, axis=1, keepdims=True)


def kernel(x):
    grid = (_COLS // _BC,)
    return pl.pallas_call(
        _scan_kernel,
        grid=grid,
        in_specs=[pl.BlockSpec((_ROWS, _BC), lambda c: (0, c))],
        out_specs=pl.BlockSpec((_ROWS, _BC), lambda c: (0, c)),
        out_shape=jax.ShapeDtypeStruct((_ROWS, _COLS), jnp.float32),
        scratch_shapes=[
            pltpu.VMEM((_ROWS, 1), jnp.float32),
            pltpu.VMEM((_NC, _BC), jnp.float32),
            pltpu.VMEM((128, 128), jnp.float32),
        ],
    )(x)
